# baseline (speedup 1.0000x reference)
import functools

import jax
import jax.numpy as jnp
from jax import lax
from jax.experimental import pallas as pl
from jax.experimental.pallas import tpu as pltpu
from jax.experimental.pallas import tpu_sc as plsc

N_TOKENS = 2048
D_MODEL = 1024
HIDDEN = 512
H2 = HIDDEN // 2
OUT_DIM = 10
OUT_PAD = 128
SC_L = 16
NUM_EXPERTS = 8
TOP_K = 2

BLK = 128
P = N_TOKENS * TOP_K + NUM_EXPERTS * BLK
NBLK = P // BLK
CHUNK = 256

SC_CORES = 2
SC_SUBCORES = 16
NW = SC_CORES * SC_SUBCORES
TPW = N_TOKENS // NW


def _router_kernel(x_ref, wr_ref, br_ref,
                   d1_ref, d2_ref, g1_ref, g2_ref, be_ref):
    x = x_ref[...]
    logits = jnp.dot(x, wr_ref[...], preferred_element_type=jnp.float32)
    logits = logits + br_ref[...]
    m = jnp.max(logits, axis=-1, keepdims=True)
    ex = jnp.exp(logits - m)
    probs = ex / jnp.sum(ex, axis=-1, keepdims=True)

    idx1 = jnp.argmax(probs, axis=-1)
    eye = lax.broadcasted_iota(jnp.int32, probs.shape, 1)
    oh1 = (eye == idx1[:, None])
    masked = jnp.where(oh1, -jnp.inf, probs)
    idx2 = jnp.argmax(masked, axis=-1)
    oh2 = (eye == idx2[:, None])
    p1 = jnp.max(probs, axis=-1)
    p2 = jnp.max(masked, axis=-1)
    denom = p1 + p2
    g1 = p1 / denom
    g2 = p2 / denom

    oh1f = oh1.astype(jnp.float32)
    oh2f = oh2.astype(jnp.float32)
    sel = oh1f + oh2f

    ii = lax.broadcasted_iota(jnp.int32, (CHUNK, CHUNK), 0)
    jj = lax.broadcasted_iota(jnp.int32, (CHUNK, CHUNK), 1)
    tstrict = (ii > jj).astype(jnp.float32)
    carry = jnp.zeros((1, NUM_EXPERTS), dtype=jnp.float32)
    chunks = []
    for c in range(N_TOKENS // CHUNK):
        blk = sel[c * CHUNK:(c + 1) * CHUNK]
        excl = jnp.dot(tstrict, blk, preferred_element_type=jnp.float32)
        chunks.append(excl + carry)
        carry = carry + jnp.sum(blk, axis=0, keepdims=True)
    exclcum = jnp.concatenate(chunks, axis=0)
    counts = carry

    ac = jnp.floor((counts + (BLK - 1)) * (1.0 / BLK))
    ei = lax.broadcasted_iota(jnp.int32, (NUM_EXPERTS, NUM_EXPERTS), 0)
    ej = lax.broadcasted_iota(jnp.int32, (NUM_EXPERTS, NUM_EXPERTS), 1)
    tinc = (ei <= ej).astype(jnp.float32)
    end_blk = jnp.dot(ac, tinc, preferred_element_type=jnp.float32)
    off_blk = end_blk - ac

    base_pos = off_blk * float(BLK) + exclcum
    dest1 = jnp.sum(oh1f * base_pos, axis=1)
    dest2 = jnp.sum(oh2f * base_pos, axis=1)
    d1_ref[...] = dest1[:, None].astype(jnp.int32)
    d2_ref[...] = dest2[:, None].astype(jnp.int32)
    g1_ref[...] = jnp.broadcast_to(g1[:, None], (N_TOKENS, OUT_PAD))
    g2_ref[...] = jnp.broadcast_to(g2[:, None], (N_TOKENS, OUT_PAD))

    bio = lax.broadcasted_iota(jnp.int32, (NBLK, NUM_EXPERTS), 0).astype(jnp.float32)
    be = jnp.sum((bio >= end_blk).astype(jnp.float32), axis=1)
    be = jnp.minimum(be, float(NUM_EXPERTS - 1))
    be_ref[...] = be[:, None].astype(jnp.int32)


def _router_meta(x, Wr, br2):
    return pl.pallas_call(
        _router_kernel,
        grid=(1,),
        in_specs=[
            pl.BlockSpec((N_TOKENS, D_MODEL), lambda i: (0, 0)),
            pl.BlockSpec((D_MODEL, NUM_EXPERTS), lambda i: (0, 0)),
            pl.BlockSpec((1, NUM_EXPERTS), lambda i: (0, 0)),
        ],
        out_specs=[
            pl.BlockSpec((N_TOKENS, 1), lambda i: (0, 0)),
            pl.BlockSpec((N_TOKENS, 1), lambda i: (0, 0)),
            pl.BlockSpec((N_TOKENS, OUT_PAD), lambda i: (0, 0)),
            pl.BlockSpec((N_TOKENS, OUT_PAD), lambda i: (0, 0)),
            pl.BlockSpec((NBLK, 1), lambda i: (0, 0)),
        ],
        out_shape=[
            jax.ShapeDtypeStruct((N_TOKENS, 1), jnp.int32),
            jax.ShapeDtypeStruct((N_TOKENS, 1), jnp.int32),
            jax.ShapeDtypeStruct((N_TOKENS, OUT_PAD), jnp.float32),
            jax.ShapeDtypeStruct((N_TOKENS, OUT_PAD), jnp.float32),
            jax.ShapeDtypeStruct((NBLK, 1), jnp.int32),
        ],
    )(x, Wr, br2)


def _sc_dispatch(x, d1, d2):
    mesh = plsc.VectorSubcoreMesh(core_axis_name="c", subcore_axis_name="s")

    @functools.partial(
        pl.kernel, mesh=mesh,
        out_type=jax.ShapeDtypeStruct((P, D_MODEL), jnp.float32),
        scratch_types=[
            pltpu.VMEM((TPW,), jnp.int32),
            pltpu.VMEM((TPW,), jnp.int32),
            pltpu.VMEM((TPW, D_MODEL), jnp.float32),
            pltpu.SemaphoreType.DMA,
        ],
    )
    def disp(x_hbm, d1_hbm, d2_hbm, xs_hbm, d1v, d2v, xv, sem):
        wid = lax.axis_index("s") * SC_CORES + lax.axis_index("c")
        base = wid * TPW
        pltpu.sync_copy(d1_hbm.at[pl.ds(base, TPW)], d1v)
        pltpu.sync_copy(d2_hbm.at[pl.ds(base, TPW)], d2v)
        pltpu.sync_copy(x_hbm.at[pl.ds(base, TPW)], xv)
        pltpu.async_copy(xv, xs_hbm.at[d1v], sem).wait()
        pltpu.async_copy(xv, xs_hbm.at[d2v], sem).wait()

    return disp(x, d1, d2)


def _expert_kernel(be_ref, x_ref, w1_ref, b1_ref, w2_ref, b2_ref,
                   w3_ref, b3_ref, out_ref):
    x = x_ref[...]
    h1 = jnp.dot(x, w1_ref[0], preferred_element_type=jnp.float32)
    h1 = jnp.maximum(h1 + b1_ref[0], 0.0)
    h2 = jnp.dot(h1, w2_ref[0], preferred_element_type=jnp.float32)
    h2 = jnp.maximum(h2 + b2_ref[0], 0.0)
    o = jnp.dot(h2, w3_ref[0], preferred_element_type=jnp.float32)
    out_ref[...] = o + b3_ref[0]


def _grouped_mlp(be, xs, W1, b1r, W2, b2r, W3p, b3p):
    grid_spec = pltpu.PrefetchScalarGridSpec(
        num_scalar_prefetch=1,
        grid=(NBLK,),
        in_specs=[
            pl.BlockSpec((BLK, D_MODEL), lambda b, be: (b, 0)),
            pl.BlockSpec((1, D_MODEL, HIDDEN), lambda b, be: (be[b], 0, 0)),
            pl.BlockSpec((1, 1, HIDDEN), lambda b, be: (be[b], 0, 0)),
            pl.BlockSpec((1, HIDDEN, H2), lambda b, be: (be[b], 0, 0)),
            pl.BlockSpec((1, 1, H2), lambda b, be: (be[b], 0, 0)),
            pl.BlockSpec((1, H2, OUT_PAD), lambda b, be: (be[b], 0, 0)),
            pl.BlockSpec((1, 1, OUT_PAD), lambda b, be: (be[b], 0, 0)),
        ],
        out_specs=pl.BlockSpec((BLK, OUT_PAD), lambda b, be: (b, 0)),
    )
    return pl.pallas_call(
        _expert_kernel,
        grid_spec=grid_spec,
        out_shape=jax.ShapeDtypeStruct((P, OUT_PAD), jnp.float32),
    )(be, xs, W1, b1r, W2, b2r, W3p, b3p)


def _sc_combine(rows, d1, d2, g1b, g2b):
    mesh = plsc.VectorSubcoreMesh(core_axis_name="c", subcore_axis_name="s")

    @functools.partial(
        pl.kernel, mesh=mesh,
        out_type=jax.ShapeDtypeStruct((N_TOKENS, OUT_PAD), jnp.float32),
        scratch_types=[
            pltpu.VMEM((TPW,), jnp.int32),
            pltpu.VMEM((TPW,), jnp.int32),
            pltpu.VMEM((TPW, OUT_PAD), jnp.float32),
            pltpu.VMEM((TPW, OUT_PAD), jnp.float32),
            pltpu.VMEM((TPW, OUT_PAD), jnp.float32),
            pltpu.VMEM((TPW, OUT_PAD), jnp.float32),
            pltpu.VMEM((TPW, OUT_PAD), jnp.float32),
            pltpu.SemaphoreType.DMA,
        ],
    )
    def comb(rows_hbm, d1_hbm, d2_hbm, g1_hbm, g2_hbm, out_hbm,
             d1v, d2v, g1v, g2v, r1v, r2v, ov, sem):
        wid = lax.axis_index("s") * SC_CORES + lax.axis_index("c")
        base = wid * TPW
        pltpu.sync_copy(d1_hbm.at[pl.ds(base, TPW)], d1v)
        pltpu.sync_copy(d2_hbm.at[pl.ds(base, TPW)], d2v)
        pltpu.sync_copy(g1_hbm.at[pl.ds(base, TPW)], g1v)
        pltpu.sync_copy(g2_hbm.at[pl.ds(base, TPW)], g2v)
        pltpu.async_copy(rows_hbm.at[d1v], r1v, sem).wait()
        pltpu.async_copy(rows_hbm.at[d2v], r2v, sem).wait()
        for i in range(TPW):
            ov[i, pl.ds(0, SC_L)] = (g1v[i, pl.ds(0, SC_L)] * r1v[i, pl.ds(0, SC_L)]
                                     + g2v[i, pl.ds(0, SC_L)] * r2v[i, pl.ds(0, SC_L)])
        pltpu.sync_copy(ov, out_hbm.at[pl.ds(base, TPW)])

    return comb(rows, d1, d2, g1b, g2b)


@jax.jit
def kernel(inputs, Wr, br, W1, b1, W2, b2, W3, b3):
    br2 = br.reshape(1, NUM_EXPERTS)
    b1r = b1[:, None, :]
    b2r = b2[:, None, :]
    W3p = jnp.pad(W3, ((0, 0), (0, 0), (0, OUT_PAD - OUT_DIM)))
    b3p = jnp.pad(b3, ((0, 0), (0, OUT_PAD - OUT_DIM)))[:, None, :]

    d1m, d2m, g1m, g2m, bem = _router_meta(inputs, Wr, br2)
    d1 = d1m.reshape(-1)
    d2 = d2m.reshape(-1)
    be = bem.reshape(-1)

    xs = _sc_dispatch(inputs, d1, d2)
    rows = _grouped_mlp(be, xs, W1, b1r, W2, b2r, W3p, b3p)
    outp = _sc_combine(rows, d1, d2, g1m, g2m)
    return outp[:, :OUT_DIM]
